# baseline (speedup 1.0000x reference)
import functools

import jax
import jax.numpy as jnp
from jax import lax
from jax.experimental import pallas as pl
from jax.experimental.pallas import tpu as pltpu
from jax.experimental.pallas import tpu_sc as plsc

_LANES = 16
_IDXBLK = 128


@functools.lru_cache(maxsize=None)
def _make_kernel(B, V, D):
    info = plsc.get_sparse_core_info()
    NC, NS = info.num_cores, info.num_subcores
    NW = NC * NS
    bpw = B // NW
    nchunk = bpw // _LANES
    nblk = (bpw * D) // _IDXBLK

    mesh = plsc.VectorSubcoreMesh(core_axis_name="c", subcore_axis_name="s")

    @functools.partial(
        pl.kernel,
        mesh=mesh,
        out_type=jax.ShapeDtypeStruct((B,), jnp.float32),
        compiler_params=pltpu.CompilerParams(needs_layout_passes=False),
        scratch_types=[
            pltpu.VMEM((bpw,), jnp.int32),
            pltpu.VMEM((bpw,), jnp.int32),
            pltpu.VMEM((bpw * D,), jnp.int32),
            pltpu.VMEM((bpw * D,), jnp.int32),
            pltpu.VMEM((bpw * D,), jnp.float32),
            pltpu.VMEM((bpw * D,), jnp.float32),
            pltpu.VMEM((bpw,), jnp.float32),
            pltpu.SemaphoreType.DMA,
        ],
    )
    def kern(uidx_hbm, iidx_hbm, uflat_hbm, iflat_hbm, out_hbm,
             uix, iix, ufidx, ifidx, uval, ival, outv, sem):
        wid = lax.axis_index("s") * NC + lax.axis_index("c")
        base = wid * bpw
        pltpu.sync_copy(uidx_hbm.at[pl.ds(base, bpw)], uix)
        pltpu.sync_copy(iidx_hbm.at[pl.ds(base, bpw)], iix)

        def build(i, carry):
            uvec = uix[pl.ds(i * _LANES, _LANES)]
            ivec = iix[pl.ds(i * _LANES, _LANES)]
            for d in range(D):
                off = d * bpw + i * _LANES
                ufidx[pl.ds(off, _LANES)] = uvec + d * V
                ifidx[pl.ds(off, _LANES)] = ivec + d * V
            return carry

        lax.fori_loop(0, nchunk, build, 0)

        copies = []
        for k in range(nblk):
            blk = pl.ds(k * _IDXBLK, _IDXBLK)
            copies.append(
                pltpu.async_copy(uflat_hbm.at[ufidx.at[blk]], uval.at[blk], sem)
            )
            copies.append(
                pltpu.async_copy(iflat_hbm.at[ifidx.at[blk]], ival.at[blk], sem)
            )
        for c in copies:
            c.wait()

        iota = lax.iota(jnp.int32, _LANES)

        def dot(i, carry):
            acc = jnp.zeros((_LANES,), jnp.float32)
            for d in range(D):
                off = d * bpw + i * _LANES
                acc = acc + uval[pl.ds(off, _LANES)] * ival[pl.ds(off, _LANES)]
            plsc.store_scatter(outv, [i * _LANES + iota], acc)
            return carry

        lax.fori_loop(0, nchunk, dot, 0)
        pltpu.sync_copy(outv, out_hbm.at[pl.ds(base, bpw)])

    return kern


@jax.jit
def kernel(behavior, user_table, item_table):
    uidx = behavior[:, 0].astype(jnp.int32)
    iidx = behavior[:, 1].astype(jnp.int32)
    V, D = user_table.shape
    uflat = user_table.T.reshape(V * D)
    iflat = item_table.T.reshape(V * D)
    return _make_kernel(behavior.shape[0], V, D)(uidx, iidx, uflat, iflat)


# speedup vs baseline: 5.5940x; 5.5940x over previous
import functools

import jax
import jax.numpy as jnp
from jax import lax
from jax.experimental import pallas as pl
from jax.experimental.pallas import tpu as pltpu
from jax.experimental.pallas import tpu_sc as plsc

_LANES = 16
_PACK = 4


@functools.lru_cache(maxsize=None)
def _make_kernel(B, V, D):
    info = plsc.get_sparse_core_info()
    NC, NS = info.num_cores, info.num_subcores
    NW = NC * NS
    bpw = B // NW
    nchunk = bpw // _LANES
    W = D * _PACK

    mesh = plsc.VectorSubcoreMesh(core_axis_name="c", subcore_axis_name="s")

    @functools.partial(
        pl.kernel,
        mesh=mesh,
        out_type=jax.ShapeDtypeStruct((B,), jnp.float32),
        compiler_params=pltpu.CompilerParams(needs_layout_passes=False),
        scratch_types=[
            pltpu.VMEM((bpw,), jnp.int32),
            pltpu.VMEM((bpw,), jnp.int32),
            pltpu.VMEM((4 * _LANES, W), jnp.float32),
            pltpu.VMEM((4 * _LANES, W), jnp.float32),
            pltpu.VMEM((bpw,), jnp.float32),
            pltpu.SemaphoreType.DMA,
        ],
    )
    def kern(uidx_hbm, iidx_hbm, utp_hbm, itp_hbm, out_hbm,
             uix, iix, urows, irows, outv, sem):
        wid = lax.axis_index("s") * NC + lax.axis_index("c")
        base = wid * bpw
        pltpu.sync_copy(uidx_hbm.at[pl.ds(base, bpw)], uix)
        pltpu.sync_copy(iidx_hbm.at[pl.ds(base, bpw)], iix)

        iota = lax.iota(jnp.int32, _LANES)

        def fire(i):
            slot = lax.bitwise_and(i, 3)
            upk = uix[pl.ds(i * _LANES, _LANES)] >> 2
            ipk = iix[pl.ds(i * _LANES, _LANES)] >> 2
            for j in range(_LANES):
                dst = pl.ds(slot * _LANES + j, 1)
                pltpu.async_copy(utp_hbm.at[pl.ds(upk[j], 1)], urows.at[dst], sem)
                pltpu.async_copy(itp_hbm.at[pl.ds(ipk[j], 1)], irows.at[dst], sem)

        def drain_compute(i):
            slot = lax.bitwise_and(i, 3)
            for j in range(_LANES):
                dst = pl.ds(slot * _LANES + j, 1)
                pltpu.make_async_copy(utp_hbm.at[pl.ds(0, 1)], urows.at[dst], sem).wait()
                pltpu.make_async_copy(utp_hbm.at[pl.ds(0, 1)], irows.at[dst], sem).wait()
            sl = pl.ds(i * _LANES, _LANES)
            srows = slot * _LANES + iota
            three = jnp.full((_LANES,), _PACK - 1, jnp.int32)
            ubase = lax.bitwise_and(uix[sl], three) * D
            ibase = lax.bitwise_and(iix[sl], three) * D
            acc = jnp.zeros((_LANES,), jnp.float32)
            for d in range(D):
                u = plsc.load_gather(urows, [srows, ubase + d])
                v = plsc.load_gather(irows, [srows, ibase + d])
                acc = acc + u * v
            plsc.store_scatter(outv, [i * _LANES + iota], acc)

        def body(i, carry):
            fire(i)

            @pl.when(i >= 2)
            def _():
                drain_compute(i - 2)

            return carry

        lax.fori_loop(0, nchunk, body, 0)
        drain_compute(nchunk - 2)
        drain_compute(nchunk - 1)
        pltpu.sync_copy(outv, out_hbm.at[pl.ds(base, bpw)])

    return kern


@jax.jit
def kernel(behavior, user_table, item_table):
    uidx = behavior[:, 0].astype(jnp.int32)
    iidx = behavior[:, 1].astype(jnp.int32)
    V, D = user_table.shape
    utp = user_table.reshape(V // _PACK, D * _PACK)
    itp = item_table.reshape(V // _PACK, D * _PACK)
    return _make_kernel(behavior.shape[0], V, D)(uidx, iidx, utp, itp)


# speedup vs baseline: 8.3552x; 1.4936x over previous
import functools

import jax
import jax.numpy as jnp
from jax import lax
from jax.experimental import pallas as pl
from jax.experimental.pallas import tpu as pltpu
from jax.experimental.pallas import tpu_sc as plsc

_LANES = 16


@functools.lru_cache(maxsize=None)
def _make_kernel(B, D):
    info = plsc.get_sparse_core_info()
    NC, NS = info.num_cores, info.num_subcores
    NW = NC * NS
    bpw = B // NW
    nchunk = bpw // _LANES

    mesh = plsc.VectorSubcoreMesh(core_axis_name="c", subcore_axis_name="s")

    @functools.partial(
        pl.kernel,
        mesh=mesh,
        out_type=jax.ShapeDtypeStruct((B,), jnp.float32),
        compiler_params=pltpu.CompilerParams(needs_layout_passes=False),
        scratch_types=[
            pltpu.VMEM((bpw,), jnp.int32),
            pltpu.VMEM((bpw,), jnp.int32),
            pltpu.VMEM((4 * _LANES, D), jnp.float32),
            pltpu.VMEM((4 * _LANES, D), jnp.float32),
            pltpu.VMEM((bpw,), jnp.float32),
            pltpu.SemaphoreType.DMA,
        ],
    )
    def kern(uidx_hbm, iidx_hbm, ut_hbm, it_hbm, out_hbm,
             uix, iix, urows, irows, outv, sem):
        wid = lax.axis_index("s") * NC + lax.axis_index("c")
        base = wid * bpw
        pltpu.sync_copy(uidx_hbm.at[pl.ds(base, bpw)], uix)
        pltpu.sync_copy(iidx_hbm.at[pl.ds(base, bpw)], iix)

        iota = lax.iota(jnp.int32, _LANES)

        def fire(i):
            slot = lax.bitwise_and(i, 3)
            uvec = uix[pl.ds(i * _LANES, _LANES)]
            ivec = iix[pl.ds(i * _LANES, _LANES)]
            for j in range(_LANES):
                dst = pl.ds(slot * _LANES + j, 1)
                pltpu.async_copy(ut_hbm.at[pl.ds(uvec[j], 1)], urows.at[dst], sem)
                pltpu.async_copy(it_hbm.at[pl.ds(ivec[j], 1)], irows.at[dst], sem)

        def drain_compute(i):
            slot = lax.bitwise_and(i, 3)
            for j in range(_LANES):
                dst = pl.ds(slot * _LANES + j, 1)
                pltpu.make_async_copy(ut_hbm.at[pl.ds(0, 1)], urows.at[dst], sem).wait()
                pltpu.make_async_copy(ut_hbm.at[pl.ds(0, 1)], irows.at[dst], sem).wait()
            srows = slot * _LANES + iota
            acc = jnp.zeros((_LANES,), jnp.float32)
            for d in range(D):
                dcol = jnp.full((_LANES,), d, jnp.int32)
                u = plsc.load_gather(urows, [srows, dcol])
                v = plsc.load_gather(irows, [srows, dcol])
                acc = acc + u * v
            plsc.store_scatter(outv, [i * _LANES + iota], acc)

        def body(i, carry):
            fire(i)

            @pl.when(i >= 2)
            def _():
                drain_compute(i - 2)

            return carry

        lax.fori_loop(0, nchunk, body, 0)
        drain_compute(nchunk - 2)
        drain_compute(nchunk - 1)
        pltpu.sync_copy(outv, out_hbm.at[pl.ds(base, bpw)])

    return kern


@jax.jit
def kernel(behavior, user_table, item_table):
    uidx = behavior[:, 0].astype(jnp.int32)
    iidx = behavior[:, 1].astype(jnp.int32)
    return _make_kernel(behavior.shape[0], item_table.shape[1])(
        uidx, iidx, user_table, item_table
    )
